# baseline (speedup 1.0000x reference)
import functools

import numpy as np
import jax
import jax.numpy as jnp
from jax import lax
from jax.experimental import pallas as pl
from jax.experimental.pallas import tpu as pltpu
from jax.experimental.pallas import tpu_sc as plsc

BATCH = 16384
D = 3200
NG = 10
NJ = 5
W = 32

_INFO = plsc.get_sparse_core_info()
_NC, _NS = _INFO.num_cores, _INFO.num_subcores
_NW = _NC * _NS
_NB128 = BATCH // 128
_BPW = _NB128 // _NW
_UNITS = _BPW * NG
_CHUNK_ROWS = NJ * 128


def _build_idx():
    wk = np.arange(_NW, dtype=np.int64)
    out = np.empty((_NW, _UNITS, _CHUNK_ROWS), dtype=np.int32)
    bl = np.arange(128, dtype=np.int64)
    for bi in range(_BPW):
        for g in range(NG):
            u = bi * NG + g
            b = (wk[:, None, None] * _BPW + bi) * 128 + bl[None, None, :]
            j = np.arange(NJ, dtype=np.int64)[None, :, None]
            k = 10 * j + g
            rho = ((b // 8) * (D // 128) + k // 4) * 32 + (b % 8) * 4 + (k % 4)
            out[:, u, :] = rho.reshape(_NW, _CHUNK_ROWS).astype(np.int32)
    return out


_IDX_NP = _build_idx()
_UNIT_LIST = [(bi, g) for bi in range(_BPW) for g in range(NG)]


def _body(in_hbm, idx_hbm, *args):
    out_hbms = args[:NG]
    idx_v, s0, s1, t0, t1 = args[NG:NG + 5]
    isem = args[NG + 5]
    gsems = args[NG + 6]
    wsems = args[NG + 7]
    sbufs = (s0, s1)
    tbufs = (t0, t1)
    wid = lax.axis_index("s") * _NC + lax.axis_index("c")
    pltpu.async_copy(idx_hbm.at[wid], idx_v, isem).wait()
    iota = lax.iota(jnp.int32, 16)

    def issue_gather(u):
        return pltpu.async_copy(
            in_hbm.at[idx_v.at[u]], sbufs[u % 2], gsems[u % 2]
        )

    gh = {0: issue_gather(0)}
    wh = {}
    for u in range(_UNITS):
        bi, g = _UNIT_LIST[u]
        p = u % 2
        if u + 1 < _UNITS:
            gh[u + 1] = issue_gather(u + 1)
        gh[u].wait()
        if u >= 2:
            wh[u - 2].wait()
        sbuf = sbufs[p]
        tbuf = tbufs[p]

        @pl.loop(0, NJ)
        def _j(j):
            rowbase = j * 128

            @pl.loop(0, 4)
            def _wq(wq):
                trow = j * 4 + wq

                @pl.loop(0, 8)
                def _ws(ws):
                    w = wq * 8 + ws
                    colv = lax.broadcast(w, (16,))
                    for l in range(8):
                        rows = rowbase + 16 * l + iota
                        v = plsc.load_gather(sbuf, [rows, colv])
                        tbuf[trow, pl.ds(ws * 128 + 16 * l, 16)] = v

        b128 = wid * _BPW + bi
        wh[u] = pltpu.async_copy(
            tbuf, out_hbms[g].at[:, b128, :], wsems[p]
        )
    wh[_UNITS - 2].wait()
    wh[_UNITS - 1].wait()


@jax.jit
def kernel(input_tensor):
    mesh = plsc.VectorSubcoreMesh(core_axis_name="c", subcore_axis_name="s")
    in_rows = (
        input_tensor.reshape(BATCH // 8, 8, D // 128, 128)
        .transpose(0, 2, 1, 3)
        .reshape(BATCH * (D // 32), W)
    )
    idx = jnp.asarray(_IDX_NP)
    out_type = tuple(
        jax.ShapeDtypeStruct((NJ * W // 8, _NB128, 1024), jnp.float32)
        for _ in range(NG)
    )
    outs = pl.kernel(
        _body,
        out_type=out_type,
        mesh=mesh,
        scratch_types=[
            pltpu.VMEM((_UNITS, _CHUNK_ROWS), jnp.int32),
            pltpu.VMEM((_CHUNK_ROWS, W), jnp.float32),
            pltpu.VMEM((_CHUNK_ROWS, W), jnp.float32),
            pltpu.VMEM((NJ * W // 8, 1024), jnp.float32),
            pltpu.VMEM((NJ * W // 8, 1024), jnp.float32),
            pltpu.SemaphoreType.DMA,
            (pltpu.SemaphoreType.DMA, pltpu.SemaphoreType.DMA),
            (pltpu.SemaphoreType.DMA, pltpu.SemaphoreType.DMA),
        ],
        compiler_params=pltpu.CompilerParams(use_tc_tiling_on_sc=False, needs_layout_passes=False),
    )(in_rows, idx)
    return tuple(
        o.reshape(NJ * W // 8, _NB128, 8, 128)
        .transpose(1, 3, 0, 2)
        .reshape(BATCH, NJ * W)
        for o in outs
    )


# speedup vs baseline: 6.7612x; 6.7612x over previous
import functools

import numpy as np
import jax
import jax.numpy as jnp
from jax import lax
from jax.experimental import pallas as pl
from jax.experimental.pallas import tpu as pltpu
from jax.experimental.pallas import tpu_sc as plsc

BATCH = 16384
D = 3200
NG = 10
NJ = 5
W = 32

_INFO = plsc.get_sparse_core_info()
_NC, _NS = _INFO.num_cores, _INFO.num_subcores
_NW = _NC * _NS
_NB128 = BATCH // 128
_BPW = _NB128 // _NW
_UNITS = _BPW * NG
_CHUNK_ROWS = NJ * 128


def _build_idx():
    wk = np.arange(_NW, dtype=np.int64)
    out = np.empty((_NW, _UNITS, _CHUNK_ROWS), dtype=np.int32)
    bl = np.arange(128, dtype=np.int64)
    for bi in range(_BPW):
        for g in range(NG):
            u = bi * NG + g
            b = (wk[:, None, None] * _BPW + bi) * 128 + bl[None, None, :]
            j = np.arange(NJ, dtype=np.int64)[None, :, None]
            k = 10 * j + g
            rho = ((b // 8) * (D // 128) + k // 4) * 32 + (b % 8) * 4 + (k % 4)
            out[:, u, :] = rho.reshape(_NW, _CHUNK_ROWS).astype(np.int32)
    return out


_IDX_NP = _build_idx()
_UNIT_LIST = [(bi, g) for bi in range(_BPW) for g in range(NG)]


def _body(in_hbm, idx_hbm, *args):
    out_hbms = args[:NG]
    idx_v, s0, s1, t0, t1 = args[NG:NG + 5]
    isem = args[NG + 5]
    gsems = args[NG + 6]
    wsems = args[NG + 7]
    sbufs = (s0, s1)
    tbufs = (t0, t1)
    wid = lax.axis_index("s") * _NC + lax.axis_index("c")
    pltpu.async_copy(idx_hbm.at[wid], idx_v, isem).wait()
    iota = lax.iota(jnp.int32, 16)

    def issue_gather(u):
        return pltpu.async_copy(
            in_hbm.at[idx_v.at[u]], sbufs[u % 2], gsems[u % 2]
        )

    row0 = lax.shift_right_logical(iota, 3)
    col0 = lax.shift_left(lax.bitwise_and(iota, 7), 7)

    gh = {0: issue_gather(0)}
    wh = {}
    for u in range(_UNITS):
        bi, g = _UNIT_LIST[u]
        p = u % 2
        if u + 1 < _UNITS:
            gh[u + 1] = issue_gather(u + 1)
        gh[u].wait()
        if u >= 2:
            wh[u - 2].wait()
        sbuf = sbufs[p]
        tbuf = tbufs[p]

        @pl.loop(0, NJ)
        def _j(j):
            rowv0 = lax.broadcast(j * 4, (16,)) + row0
            rowv1 = rowv0 + 2
            rbase = j * 128

            @functools.partial(
                plsc.parallel_loop, 0, 128, unroll=8,
                carry=col0,
            )
            def _bl(bl, colv):
                r = rbase + bl
                v0 = sbuf[r, pl.ds(0, 16)]
                v1 = sbuf[r, pl.ds(16, 16)]
                plsc.store_scatter(tbuf, [rowv0, colv], v0)
                plsc.store_scatter(tbuf, [rowv1, colv], v1)
                return colv + 1

        b128 = wid * _BPW + bi
        wh[u] = pltpu.async_copy(
            tbuf, out_hbms[g].at[:, b128, :], wsems[p]
        )
    wh[_UNITS - 2].wait()
    wh[_UNITS - 1].wait()


@jax.jit
def kernel(input_tensor):
    mesh = plsc.VectorSubcoreMesh(core_axis_name="c", subcore_axis_name="s")
    in_rows = (
        input_tensor.reshape(BATCH // 8, 8, D // 128, 128)
        .transpose(0, 2, 1, 3)
        .reshape(BATCH * (D // 32), W)
    )
    idx = jnp.asarray(_IDX_NP)
    out_type = tuple(
        jax.ShapeDtypeStruct((NJ * W // 8, _NB128, 1024), jnp.float32)
        for _ in range(NG)
    )
    outs = pl.kernel(
        _body,
        out_type=out_type,
        mesh=mesh,
        scratch_types=[
            pltpu.VMEM((_UNITS, _CHUNK_ROWS), jnp.int32),
            pltpu.VMEM((_CHUNK_ROWS, W), jnp.float32),
            pltpu.VMEM((_CHUNK_ROWS, W), jnp.float32),
            pltpu.VMEM((NJ * W // 8, 1024), jnp.float32),
            pltpu.VMEM((NJ * W // 8, 1024), jnp.float32),
            pltpu.SemaphoreType.DMA,
            (pltpu.SemaphoreType.DMA, pltpu.SemaphoreType.DMA),
            (pltpu.SemaphoreType.DMA, pltpu.SemaphoreType.DMA),
        ],
        compiler_params=pltpu.CompilerParams(use_tc_tiling_on_sc=False, needs_layout_passes=False),
    )(in_rows, idx)
    return tuple(
        o.reshape(NJ * W // 8, _NB128, 8, 128)
        .transpose(1, 3, 0, 2)
        .reshape(BATCH, NJ * W)
        for o in outs
    )
